# Initial kernel scaffold; baseline (speedup 1.0000x reference)
import jax
import jax.numpy as jnp
from jax.experimental import pallas as pl


def kernel(x1, x2, x3, batch, scales):
    raise NotImplementedError("write your pallas kernel here")


# speedup vs baseline: 4.5690x; 4.5690x over previous
import functools

import jax
import jax.numpy as jnp
from jax import lax
from jax.experimental import pallas as pl
from jax.experimental.pallas import tpu as pltpu
from jax.experimental.pallas import tpu_sc as plsc

_N = 320000
_D = 128
_S = 1024
_C = 128
_NCHUNK = _N // _C
_NC = 2
_NS = 16
_NW = _NC * _NS
_RPT = _S // _NS


def _sc_segment_sum(x1, x2, x3, batch, scalesb, zrows):
    mesh = plsc.VectorSubcoreMesh(core_axis_name="c", subcore_axis_name="s")

    @functools.partial(
        pl.kernel,
        mesh=mesh,
        out_type=jax.ShapeDtypeStruct((_NC, _S, _D), jnp.float32),
        scratch_types=[
            pltpu.VMEM((_C, _D), jnp.float32),
            pltpu.VMEM((_C,), jnp.int32),
            pltpu.VMEM((_RPT, _D), jnp.float32),
            pltpu.VMEM((_RPT, _D), jnp.float32),
            pltpu.VMEM((_RPT, _D), jnp.float32),
            pltpu.VMEM((3, 16), jnp.float32),
            pltpu.VMEM_SHARED((_S, _D), jnp.float32),
            pltpu.VMEM_SHARED((_S, _D), jnp.float32),
            pltpu.VMEM_SHARED((_S, _D), jnp.float32),
        ],
    )
    def body(x1h, x2h, x3h, bh, sclh, zh, outh,
             rows_v, idx_v, t1_v, t2_v, t3_v, scl_v, acc1, acc2, acc3):
        cid = lax.axis_index("c")
        sid = lax.axis_index("s")
        wid = sid * _NC + cid

        pltpu.sync_copy(zh, t1_v)
        pltpu.sync_copy(t1_v, acc1.at[pl.ds(sid * _RPT, _RPT)])
        pltpu.sync_copy(t1_v, acc2.at[pl.ds(sid * _RPT, _RPT)])
        pltpu.sync_copy(t1_v, acc3.at[pl.ds(sid * _RPT, _RPT)])
        plsc.subcore_barrier()

        nb = (_NCHUNK - wid + _NW - 1) // _NW

        def chunk_body(i, carry):
            c = wid + i * _NW
            base = c * _C
            pltpu.sync_copy(bh.at[pl.ds(base, _C)], idx_v)
            pltpu.sync_copy(x1h.at[pl.ds(base, _C)], rows_v)
            pltpu.sync_copy(rows_v, acc1.at[idx_v], add=True)
            pltpu.sync_copy(x2h.at[pl.ds(base, _C)], rows_v)
            pltpu.sync_copy(rows_v, acc2.at[idx_v], add=True)
            pltpu.sync_copy(x3h.at[pl.ds(base, _C)], rows_v)
            pltpu.sync_copy(rows_v, acc3.at[idx_v], add=True)
            return carry

        lax.fori_loop(0, nb, chunk_body, 0)
        plsc.subcore_barrier()

        r0 = sid * _RPT
        pltpu.sync_copy(acc1.at[pl.ds(r0, _RPT)], t1_v)
        pltpu.sync_copy(acc2.at[pl.ds(r0, _RPT)], t2_v)
        pltpu.sync_copy(acc3.at[pl.ds(r0, _RPT)], t3_v)
        pltpu.sync_copy(sclh, scl_v)
        s1 = scl_v[0]
        s2 = scl_v[1]
        s3 = scl_v[2]

        def row_body(r, carry):
            for j in range(_D // 16):
                sl = pl.ds(j * 16, 16)
                v = t1_v[r, sl] * s1 + t2_v[r, sl] * s2 + t3_v[r, sl] * s3
                t1_v[r, sl] = v
            return carry

        lax.fori_loop(0, _RPT, row_body, 0)
        pltpu.sync_copy(t1_v, outh.at[cid].at[pl.ds(r0, _RPT)])

    return body(x1, x2, x3, batch, scalesb, zrows)


def _tc_add(partials):
    def body(p_ref, o_ref):
        o_ref[...] = p_ref[0] + p_ref[1]

    return pl.pallas_call(
        body,
        out_shape=jax.ShapeDtypeStruct((_S, _D), jnp.float32),
    )(partials)


def kernel(x1, x2, x3, batch, scales):
    batch_i = batch.astype(jnp.int32)
    scalesb = jnp.broadcast_to(
        scales.astype(jnp.float32)[:, None], (3, 16))
    zrows = jnp.zeros((_RPT, _D), jnp.float32)
    partials = _sc_segment_sum(x1, x2, x3, batch_i, scalesb, zrows)
    return _tc_add(partials)


# speedup vs baseline: 5.0825x; 1.1124x over previous
import functools

import jax
import jax.numpy as jnp
from jax import lax
from jax.experimental import pallas as pl
from jax.experimental.pallas import tpu as pltpu
from jax.experimental.pallas import tpu_sc as plsc

_N = 320000
_D = 128
_S = 1024
_CI = 128
_C = 128
_NROW = _N // _CI
_NCHUNK = _N // _C
_NC = 2
_NS = 16
_NW = _NC * _NS
_CPW = _NCHUNK // _NW
_XTRA = _NCHUNK - _CPW * _NW
_IPW = _CPW + 1 + 9
_RPT = _S // _NS


def _sc_segment_sum(x1, x2, x3, batch2d, scalesb):
    mesh = plsc.VectorSubcoreMesh(core_axis_name="c", subcore_axis_name="s")

    @functools.partial(
        pl.kernel,
        mesh=mesh,
        out_type=jax.ShapeDtypeStruct((_NC, _S, _D), jnp.float32),
        scratch_types=[
            pltpu.VMEM((_C, _D), jnp.float32),
            pltpu.VMEM((_C, _D), jnp.float32),
            pltpu.VMEM((_C, _D), jnp.float32),
            pltpu.VMEM((_IPW, _CI), jnp.int32),
            pltpu.VMEM((3, 16), jnp.float32),
            pltpu.VMEM_SHARED((_S, _D), jnp.float32),
            pltpu.VMEM_SHARED((_S, _D), jnp.float32),
            pltpu.VMEM_SHARED((_S, _D), jnp.float32),
            pltpu.SemaphoreType.DMA,
            pltpu.SemaphoreType.DMA,
        ],
    )
    def body(x1h, x2h, x3h, bh, sclh, outh,
             r1_v, r2_v, r3_v, idx_v, scl_v, acc1, acc2, acc3, lsem, ssem):
        cid = lax.axis_index("c")
        sid = lax.axis_index("s")
        wid = sid * _NC + cid

        def zrow_body(r, carry):
            for j in range(_D // 16):
                r1_v[r, pl.ds(j * 16, 16)] = jnp.zeros((16,), jnp.float32)
            return carry

        lax.fori_loop(0, _RPT, zrow_body, 0)
        z64 = r1_v.at[pl.ds(0, _RPT)]
        pltpu.sync_copy(z64, acc1.at[pl.ds(sid * _RPT, _RPT)])
        pltpu.sync_copy(z64, acc2.at[pl.ds(sid * _RPT, _RPT)])
        pltpu.sync_copy(z64, acc3.at[pl.ds(sid * _RPT, _RPT)])

        s_w = wid * _CPW + jnp.minimum(wid, _XTRA)
        nb = _CPW + jnp.where(wid < _XTRA, 1, 0)
        abase = s_w // 8 * 8
        ioff = s_w - abase
        pltpu.sync_copy(bh.at[pl.ds(abase, _IPW)], idx_v)
        plsc.subcore_barrier()

        def chunk_body(k, carry):
            base = (s_w + k) * _C
            h1 = pltpu.async_copy(x1h.at[pl.ds(base, _C)], r1_v, lsem)
            h2 = pltpu.async_copy(x2h.at[pl.ds(base, _C)], r2_v, lsem)
            h3 = pltpu.async_copy(x3h.at[pl.ds(base, _C)], r3_v, lsem)
            h1.wait()
            h2.wait()
            h3.wait()
            ws = []
            for rv, acc in ((r1_v, acc1), (r2_v, acc2), (r3_v, acc3)):
                ws.append(pltpu.async_copy(
                    rv, acc.at[idx_v.at[ioff + k]], ssem, add=True))
            for w in ws:
                w.wait()
            return carry

        lax.fori_loop(0, nb, chunk_body, 0)
        plsc.subcore_barrier()

        r0 = sid * _RPT
        pltpu.sync_copy(acc1.at[pl.ds(r0, _RPT)], r1_v.at[pl.ds(0, _RPT)])
        pltpu.sync_copy(acc2.at[pl.ds(r0, _RPT)], r2_v.at[pl.ds(0, _RPT)])
        pltpu.sync_copy(acc3.at[pl.ds(r0, _RPT)], r3_v.at[pl.ds(0, _RPT)])
        pltpu.sync_copy(sclh, scl_v)
        s1 = scl_v[0]
        s2 = scl_v[1]
        s3 = scl_v[2]

        def row_body(r, carry):
            for j in range(_D // 16):
                sl = pl.ds(j * 16, 16)
                v = r1_v[r, sl] * s1 + r2_v[r, sl] * s2 + r3_v[r, sl] * s3
                r1_v[r, sl] = v
            return carry

        lax.fori_loop(0, _RPT, row_body, 0)
        pltpu.sync_copy(r1_v.at[pl.ds(0, _RPT)], outh.at[cid].at[pl.ds(r0, _RPT)])

    return body(x1, x2, x3, batch2d, scalesb)


def _tc_add(partials):
    def body(p_ref, o_ref):
        o_ref[...] = p_ref[0] + p_ref[1]

    return pl.pallas_call(
        body,
        out_shape=jax.ShapeDtypeStruct((_S, _D), jnp.float32),
    )(partials)


def kernel(x1, x2, x3, batch, scales):
    batch_i = batch.astype(jnp.int32)
    batch2d = jnp.pad(batch_i.reshape(_NROW, _CI),
                      ((0, _IPW), (0, 0)))
    scalesb = jnp.broadcast_to(
        scales.astype(jnp.float32)[:, None], (3, 16))
    partials = _sc_segment_sum(x1, x2, x3, batch2d, scalesb)
    return _tc_add(partials)


# speedup vs baseline: 5.7809x; 1.1374x over previous
import functools

import jax
import jax.numpy as jnp
from jax import lax
from jax.experimental import pallas as pl
from jax.experimental.pallas import tpu as pltpu
from jax.experimental.pallas import tpu_sc as plsc

_N = 320000
_D = 128
_S = 1024
_C = 128
_NCHUNK = _N // _C
_NC = 2
_NS = 16
_NW = _NC * _NS
_CPW = _NCHUNK // _NW
_XTRA = _NCHUNK - _CPW * _NW
_NPAIR = _CPW // 2
_RPT = _S // _NS


def _sc_segment_sum(x1, x2, x3, batch, scalesb):
    mesh = plsc.VectorSubcoreMesh(core_axis_name="c", subcore_axis_name="s")

    @functools.partial(
        pl.kernel,
        mesh=mesh,
        out_type=jax.ShapeDtypeStruct((_NC, _S, _D), jnp.float32),
        scratch_types=[
            pltpu.VMEM((_C, _D), jnp.float32),
            pltpu.VMEM((_C, _D), jnp.float32),
            pltpu.VMEM((_C, _D), jnp.float32),
            pltpu.VMEM((_C,), jnp.int32),
            pltpu.VMEM((_C, _D), jnp.float32),
            pltpu.VMEM((_C, _D), jnp.float32),
            pltpu.VMEM((_C, _D), jnp.float32),
            pltpu.VMEM((_C,), jnp.int32),
            pltpu.VMEM((3, 16), jnp.float32),
            pltpu.VMEM_SHARED((_S, _D), jnp.float32),
            pltpu.VMEM_SHARED((_S, _D), jnp.float32),
            pltpu.VMEM_SHARED((_S, _D), jnp.float32),
            pltpu.SemaphoreType.DMA,
            pltpu.SemaphoreType.DMA,
            pltpu.SemaphoreType.DMA,
            pltpu.SemaphoreType.DMA,
        ],
    )
    def body(x1h, x2h, x3h, bh, sclh, outh,
             r1a, r2a, r3a, ixa, r1b, r2b, r3b, ixb, scl_v,
             acc1, acc2, acc3, lsa, lsb, ssa, ssb):
        cid = lax.axis_index("c")
        sid = lax.axis_index("s")
        wid = sid * _NC + cid
        bufs_a = (r1a, r2a, r3a, ixa)
        bufs_b = (r1b, r2b, r3b, ixb)

        def issue_loads(c, bufs, sem):
            base = c * _C
            r1, r2, r3, ix = bufs
            pltpu.async_copy(bh.at[pl.ds(base, _C)], ix, sem)
            pltpu.async_copy(x1h.at[pl.ds(base, _C)], r1, sem)
            pltpu.async_copy(x2h.at[pl.ds(base, _C)], r2, sem)
            pltpu.async_copy(x3h.at[pl.ds(base, _C)], r3, sem)

        def drain_loads(bufs, sem):
            r1, r2, r3, ix = bufs
            pltpu.make_async_copy(bh.at[pl.ds(0, _C)], ix, sem).wait()
            pltpu.make_async_copy(x1h.at[pl.ds(0, _C)], r1, sem).wait()
            pltpu.make_async_copy(x2h.at[pl.ds(0, _C)], r2, sem).wait()
            pltpu.make_async_copy(x3h.at[pl.ds(0, _C)], r3, sem).wait()

        def issue_scats(bufs, sem):
            r1, r2, r3, ix = bufs
            pltpu.async_copy(r1, acc1.at[ix], sem, add=True)
            pltpu.async_copy(r2, acc2.at[ix], sem, add=True)
            pltpu.async_copy(r3, acc3.at[ix], sem, add=True)

        def drain_scats(bufs, sem):
            r1, r2, r3, ix = bufs
            pltpu.make_async_copy(r1, acc1.at[pl.ds(0, _C)], sem).wait()
            pltpu.make_async_copy(r2, acc2.at[pl.ds(0, _C)], sem).wait()
            pltpu.make_async_copy(r3, acc3.at[pl.ds(0, _C)], sem).wait()

        def zrow_body(r, carry):
            for j in range(_D // 16):
                r1a[r, pl.ds(j * 16, 16)] = jnp.zeros((16,), jnp.float32)
            return carry

        lax.fori_loop(0, _RPT, zrow_body, 0)
        z64 = r1a.at[pl.ds(0, _RPT)]
        pltpu.sync_copy(z64, acc1.at[pl.ds(sid * _RPT, _RPT)])
        pltpu.sync_copy(z64, acc2.at[pl.ds(sid * _RPT, _RPT)])
        pltpu.sync_copy(z64, acc3.at[pl.ds(sid * _RPT, _RPT)])
        plsc.subcore_barrier()

        s_w = wid * _CPW + jnp.minimum(wid, _XTRA)

        issue_loads(s_w, bufs_a, lsa)

        def pair_body(p, carry):
            c0 = s_w + 2 * p

            @pl.when(p > 0)
            def _():
                drain_scats(bufs_b, ssb)

            issue_loads(c0 + 1, bufs_b, lsb)
            drain_loads(bufs_a, lsa)
            issue_scats(bufs_a, ssa)

            @pl.when(p < _NPAIR - 1)
            def _():
                drain_scats(bufs_a, ssa)
                issue_loads(c0 + 2, bufs_a, lsa)

            drain_loads(bufs_b, lsb)
            issue_scats(bufs_b, ssb)
            return carry

        lax.fori_loop(0, _NPAIR, pair_body, 0)
        drain_scats(bufs_a, ssa)
        drain_scats(bufs_b, ssb)

        @pl.when(wid < _XTRA)
        def _():
            issue_loads(s_w + _CPW, bufs_a, lsa)
            drain_loads(bufs_a, lsa)
            issue_scats(bufs_a, ssa)
            drain_scats(bufs_a, ssa)

        plsc.subcore_barrier()

        r0 = sid * _RPT
        pltpu.sync_copy(acc1.at[pl.ds(r0, _RPT)], r1a.at[pl.ds(0, _RPT)])
        pltpu.sync_copy(acc2.at[pl.ds(r0, _RPT)], r2a.at[pl.ds(0, _RPT)])
        pltpu.sync_copy(acc3.at[pl.ds(r0, _RPT)], r3a.at[pl.ds(0, _RPT)])
        pltpu.sync_copy(sclh, scl_v)
        s1 = scl_v[0]
        s2 = scl_v[1]
        s3 = scl_v[2]

        def row_body(r, carry):
            for j in range(_D // 16):
                sl = pl.ds(j * 16, 16)
                v = r1a[r, sl] * s1 + r2a[r, sl] * s2 + r3a[r, sl] * s3
                r1a[r, sl] = v
            return carry

        lax.fori_loop(0, _RPT, row_body, 0)
        pltpu.sync_copy(r1a.at[pl.ds(0, _RPT)],
                        outh.at[cid].at[pl.ds(r0, _RPT)])

    return body(x1, x2, x3, batch, scalesb)


def _tc_add(partials):
    def body(p_ref, o_ref):
        o_ref[...] = p_ref[0] + p_ref[1]

    return pl.pallas_call(
        body,
        out_shape=jax.ShapeDtypeStruct((_S, _D), jnp.float32),
    )(partials)


def kernel(x1, x2, x3, batch, scales):
    batch_i = batch.astype(jnp.int32)
    scalesb = jnp.broadcast_to(
        scales.astype(jnp.float32)[:, None], (3, 16))
    partials = _sc_segment_sum(x1, x2, x3, batch_i, scalesb)
    return _tc_add(partials)


# speedup vs baseline: 8.8522x; 1.5313x over previous
import functools

import jax
import jax.numpy as jnp
from jax import lax
from jax.experimental import pallas as pl
from jax.experimental.pallas import tpu as pltpu
from jax.experimental.pallas import tpu_sc as plsc

_N = 320000
_D = 128
_S = 1024
_C = 128
_NCHUNK = _N // _C
_NC = 2
_NS = 16
_NW = _NC * _NS
_CPW = _NCHUNK // _NW
_XTRA = _NCHUNK - _CPW * _NW
_NPAIR = _CPW // 2
_RPT = _S // _NS


def _sc_segment_sum(x1, x2, x3, batch, scalesb):
    mesh = plsc.VectorSubcoreMesh(core_axis_name="c", subcore_axis_name="s")

    @functools.partial(
        pl.kernel,
        mesh=mesh,
        out_type=jax.ShapeDtypeStruct((_NC, _S, _D), jnp.float32),
        scratch_types=[
            pltpu.VMEM((_C, _D), jnp.float32),
            pltpu.VMEM((_C, _D), jnp.float32),
            pltpu.VMEM((_C, _D), jnp.float32),
            pltpu.VMEM((_C,), jnp.int32),
            pltpu.VMEM((_C, _D), jnp.float32),
            pltpu.VMEM((_C, _D), jnp.float32),
            pltpu.VMEM((_C, _D), jnp.float32),
            pltpu.VMEM((_C,), jnp.int32),
            pltpu.VMEM((3, 16), jnp.float32),
            pltpu.VMEM_SHARED((_S, _D), jnp.float32),
            pltpu.SemaphoreType.DMA,
            pltpu.SemaphoreType.DMA,
            pltpu.SemaphoreType.DMA,
            pltpu.SemaphoreType.DMA,
        ],
    )
    def body(x1h, x2h, x3h, bh, sclh, outh,
             r1a, r2a, r3a, ixa, r1b, r2b, r3b, ixb, scl_v,
             acc, lsa, lsb, ssa, ssb):
        cid = lax.axis_index("c")
        sid = lax.axis_index("s")
        wid = sid * _NC + cid
        bufs_a = (r1a, r2a, r3a, ixa)
        bufs_b = (r1b, r2b, r3b, ixb)

        def issue_loads(c, bufs, sem):
            base = c * _C
            r1, r2, r3, ix = bufs
            pltpu.async_copy(bh.at[pl.ds(base, _C)], ix, sem)
            pltpu.async_copy(x1h.at[pl.ds(base, _C)], r1, sem)
            pltpu.async_copy(x2h.at[pl.ds(base, _C)], r2, sem)
            pltpu.async_copy(x3h.at[pl.ds(base, _C)], r3, sem)

        def drain_loads(bufs, sem):
            r1, r2, r3, ix = bufs
            pltpu.make_async_copy(bh.at[pl.ds(0, _C)], ix, sem).wait()
            pltpu.make_async_copy(x1h.at[pl.ds(0, _C)], r1, sem).wait()
            pltpu.make_async_copy(x2h.at[pl.ds(0, _C)], r2, sem).wait()
            pltpu.make_async_copy(x3h.at[pl.ds(0, _C)], r3, sem).wait()

        def combine(bufs):
            r1, r2, r3, ix = bufs
            s1 = scl_v[0]
            s2 = scl_v[1]
            s3 = scl_v[2]

            def row_body(r, carry):
                for j in range(_D // 16):
                    sl = pl.ds(j * 16, 16)
                    r1[r, sl] = (r1[r, sl] * s1 + r2[r, sl] * s2
                                 + r3[r, sl] * s3)
                return carry

            lax.fori_loop(0, _C, row_body, 0)

        def issue_scat(bufs, sem):
            r1, r2, r3, ix = bufs
            pltpu.async_copy(r1, acc.at[ix], sem, add=True)

        def drain_scat(bufs, sem):
            r1, r2, r3, ix = bufs
            pltpu.make_async_copy(r1, acc.at[pl.ds(0, _C)], sem).wait()

        def zrow_body(r, carry):
            for j in range(_D // 16):
                r1a[r, pl.ds(j * 16, 16)] = jnp.zeros((16,), jnp.float32)
            return carry

        lax.fori_loop(0, _RPT, zrow_body, 0)
        pltpu.sync_copy(r1a.at[pl.ds(0, _RPT)],
                        acc.at[pl.ds(sid * _RPT, _RPT)])
        pltpu.sync_copy(sclh, scl_v)
        plsc.subcore_barrier()

        s_w = wid * _CPW + jnp.minimum(wid, _XTRA)

        issue_loads(s_w, bufs_a, lsa)

        def pair_body(p, carry):
            c0 = s_w + 2 * p

            @pl.when(p > 0)
            def _():
                drain_scat(bufs_b, ssb)

            issue_loads(c0 + 1, bufs_b, lsb)
            drain_loads(bufs_a, lsa)
            combine(bufs_a)
            issue_scat(bufs_a, ssa)

            @pl.when(p < _NPAIR - 1)
            def _():
                drain_scat(bufs_a, ssa)
                issue_loads(c0 + 2, bufs_a, lsa)

            drain_loads(bufs_b, lsb)
            combine(bufs_b)
            issue_scat(bufs_b, ssb)
            return carry

        lax.fori_loop(0, _NPAIR, pair_body, 0)
        drain_scat(bufs_a, ssa)
        drain_scat(bufs_b, ssb)

        @pl.when(wid < _XTRA)
        def _():
            issue_loads(s_w + _CPW, bufs_a, lsa)
            drain_loads(bufs_a, lsa)
            combine(bufs_a)
            issue_scat(bufs_a, ssa)
            drain_scat(bufs_a, ssa)

        plsc.subcore_barrier()

        r0 = sid * _RPT
        pltpu.sync_copy(acc.at[pl.ds(r0, _RPT)], r1a.at[pl.ds(0, _RPT)])
        pltpu.sync_copy(r1a.at[pl.ds(0, _RPT)],
                        outh.at[cid].at[pl.ds(r0, _RPT)])

    return body(x1, x2, x3, batch, scalesb)


def _tc_add(partials):
    def body(p_ref, o_ref):
        o_ref[...] = p_ref[0] + p_ref[1]

    return pl.pallas_call(
        body,
        out_shape=jax.ShapeDtypeStruct((_S, _D), jnp.float32),
    )(partials)


def kernel(x1, x2, x3, batch, scales):
    batch_i = batch.astype(jnp.int32)
    scalesb = jnp.broadcast_to(
        scales.astype(jnp.float32)[:, None], (3, 16))
    partials = _sc_segment_sum(x1, x2, x3, batch_i, scalesb)
    return _tc_add(partials)
